# Initial kernel scaffold; baseline (speedup 1.0000x reference)
import jax
import jax.numpy as jnp
from jax.experimental import pallas as pl


def kernel(positions, W1, b1, W2, b2, W3, b3):
    raise NotImplementedError("write your pallas kernel here")


# speedup vs baseline: 1.1827x; 1.1827x over previous
import jax
import jax.numpy as jnp
import numpy as np
from jax.experimental import pallas as pl
from jax.experimental.pallas import tpu as pltpu

N_RBF = 16
N_HIDDEN = 32
CUTOFF = 5.0
N_ATOMS = 2048
BLOCK = 256
NPROG = N_ATOMS // BLOCK


def _centers():
    c = np.linspace(0.0, CUTOFF, N_RBF, dtype=np.float32)
    width = c[1] - c[0]
    coeff = -0.5 / (width * width)
    return c, np.float32(coeff)


def _body(rowpos_ref, posT_ref, W1T_ref, b1_ref, W2T_ref, b2_ref,
          W3T_ref, b3_ref, out_ref):
    centers, coeff = _centers()

    cx = rowpos_ref[:, 0:1]
    cy = rowpos_ref[:, 1:2]
    cz = rowpos_ref[:, 2:3]
    rx = posT_ref[0:1, :]
    ry = posT_ref[1:2, :]
    rz = posT_ref[2:3, :]

    dx = cx - rx
    dy = cy - ry
    dz = cz - rz
    d2 = dx * dx + dy * dy + dz * dz
    dist = jnp.sqrt(d2)
    mask = (d2 > 0.0) & (dist < CUTOFF)

    rows = []
    for k in range(N_RBF):
        t = dist - centers[k]
        r = jnp.where(mask, jnp.exp(coeff * (t * t)), 0.0)
        rows.append(jnp.sum(r, axis=1))
    feat = jnp.stack(rows, axis=0)

    h = jax.lax.dot_general(W1T_ref[...], feat, (((1,), (0,)), ((), ())),
                            preferred_element_type=jnp.float32)
    h = jax.nn.silu(h + b1_ref[...])
    h = jax.lax.dot_general(W2T_ref[...], h, (((1,), (0,)), ((), ())),
                            preferred_element_type=jnp.float32)
    h = jax.nn.silu(h + b2_ref[...])
    e = jax.lax.dot_general(W3T_ref[...], h, (((1,), (0,)), ((), ())),
                            preferred_element_type=jnp.float32)
    energy = jnp.sum(e + b3_ref[...])
    out_ref[...] = jnp.broadcast_to(energy, (1, 1, 128))


def kernel(positions, W1, b1, W2, b2, W3, b3):
    f32 = jnp.float32
    positions = positions.astype(f32)
    rowpos = jnp.zeros((N_ATOMS, 128), f32).at[:, :3].set(positions)
    posT = jnp.zeros((8, N_ATOMS), f32).at[:3, :].set(positions.T)

    W1T = W1.T.astype(f32)
    W2T = W2.T.astype(f32)
    W3T = W3.T.astype(f32)
    b1c = b1.astype(f32)[:, None]
    b2c = b2.astype(f32)[:, None]
    b3c = b3.astype(f32)[:, None]

    out = pl.pallas_call(
        _body,
        grid=(NPROG,),
        in_specs=[
            pl.BlockSpec((BLOCK, 128), lambda i: (i, 0)),
            pl.BlockSpec((8, N_ATOMS), lambda i: (0, 0)),
            pl.BlockSpec(W1T.shape, lambda i: (0, 0)),
            pl.BlockSpec(b1c.shape, lambda i: (0, 0)),
            pl.BlockSpec(W2T.shape, lambda i: (0, 0)),
            pl.BlockSpec(b2c.shape, lambda i: (0, 0)),
            pl.BlockSpec(W3T.shape, lambda i: (0, 0)),
            pl.BlockSpec(b3c.shape, lambda i: (0, 0)),
        ],
        out_specs=pl.BlockSpec((1, 1, 128), lambda i: (i, 0, 0)),
        out_shape=jax.ShapeDtypeStruct((NPROG, 1, 128), f32),
        compiler_params=pltpu.CompilerParams(
            dimension_semantics=("parallel",),
        ),
    )(rowpos, posT, W1T, b1c, W2T, b2c, W3T, b3c)
    return jnp.sum(out[:, 0, 0])


# speedup vs baseline: 1.7610x; 1.4890x over previous
import jax
import jax.numpy as jnp
import numpy as np
from jax.experimental import pallas as pl
from jax.experimental.pallas import tpu as pltpu

N_RBF = 16
N_HIDDEN = 32
CUTOFF = 5.0
N_ATOMS = 2048
BLOCK = 256
NPROG = N_ATOMS // BLOCK


def _rbf_consts():
    c = np.linspace(0.0, CUTOFF, N_RBF, dtype=np.float32).astype(np.float64)
    width32 = np.float32(np.float32(c[1]) - np.float32(c[0]))
    coeff = np.float64(np.float32(-0.5 / (width32 * width32)))
    w = (c[N_RBF - 1] - c[0]) / (N_RBF - 1)
    a = -2.0 * coeff * w
    bases = [c[0], c[8]]
    scales = np.array(
        [np.exp(coeff * (c[k] - bases[k // 8]) ** 2) for k in range(N_RBF)],
        dtype=np.float64,
    )
    uclamp = 87.0 / (7.0 * a)
    return (np.float32(coeff), np.float32(a), np.float32(uclamp),
            [np.float32(b) for b in bases], scales.astype(np.float32))


def _body(rowpos_ref, posT_ref, W1T_ref, b1_ref, W2T_ref, b2_ref,
          W3T_ref, b3_ref, out_ref):
    coeff, a, uclamp, bases, scales = _rbf_consts()

    cx = rowpos_ref[:, 0:1]
    cy = rowpos_ref[:, 1:2]
    cz = rowpos_ref[:, 2:3]
    rx = posT_ref[0:1, :]
    ry = posT_ref[1:2, :]
    rz = posT_ref[2:3, :]

    dx = cx - rx
    dy = cy - ry
    dz = cz - rz
    d2 = dx * dx + dy * dy + dz * dz
    dist = jnp.sqrt(d2)
    mask = (d2 > 0.0) & (dist < CUTOFF)

    rows = []
    for g in range(2):
        u = jnp.minimum(dist - bases[g], uclamp)
        s = jnp.where(mask, jnp.exp(coeff * (u * u)), 0.0)
        t = jnp.exp(a * u)
        p = s
        rows.append(jnp.sum(p, axis=1) * float(scales[8 * g]))
        for m in range(1, 8):
            p = p * t
            rows.append(jnp.sum(p, axis=1) * float(scales[8 * g + m]))
    feat = jnp.stack(rows, axis=0)

    h = jax.lax.dot_general(W1T_ref[...], feat, (((1,), (0,)), ((), ())),
                            preferred_element_type=jnp.float32)
    h = jax.nn.silu(h + b1_ref[...])
    h = jax.lax.dot_general(W2T_ref[...], h, (((1,), (0,)), ((), ())),
                            preferred_element_type=jnp.float32)
    h = jax.nn.silu(h + b2_ref[...])
    e = jax.lax.dot_general(W3T_ref[...], h, (((1,), (0,)), ((), ())),
                            preferred_element_type=jnp.float32)
    energy = jnp.sum(e + b3_ref[...])
    out_ref[...] = jnp.broadcast_to(energy, (1, 1, 128))


def kernel(positions, W1, b1, W2, b2, W3, b3):
    f32 = jnp.float32
    positions = positions.astype(f32)
    rowpos = jnp.zeros((N_ATOMS, 128), f32).at[:, :3].set(positions)
    posT = jnp.zeros((8, N_ATOMS), f32).at[:3, :].set(positions.T)

    W1T = W1.T.astype(f32)
    W2T = W2.T.astype(f32)
    W3T = W3.T.astype(f32)
    b1c = b1.astype(f32)[:, None]
    b2c = b2.astype(f32)[:, None]
    b3c = b3.astype(f32)[:, None]

    out = pl.pallas_call(
        _body,
        grid=(NPROG,),
        in_specs=[
            pl.BlockSpec((BLOCK, 128), lambda i: (i, 0)),
            pl.BlockSpec((8, N_ATOMS), lambda i: (0, 0)),
            pl.BlockSpec(W1T.shape, lambda i: (0, 0)),
            pl.BlockSpec(b1c.shape, lambda i: (0, 0)),
            pl.BlockSpec(W2T.shape, lambda i: (0, 0)),
            pl.BlockSpec(b2c.shape, lambda i: (0, 0)),
            pl.BlockSpec(W3T.shape, lambda i: (0, 0)),
            pl.BlockSpec(b3c.shape, lambda i: (0, 0)),
        ],
        out_specs=pl.BlockSpec((1, 1, 128), lambda i: (i, 0, 0)),
        out_shape=jax.ShapeDtypeStruct((NPROG, 1, 128), f32),
        compiler_params=pltpu.CompilerParams(
            dimension_semantics=("parallel",),
        ),
    )(rowpos, posT, W1T, b1c, W2T, b2c, W3T, b3c)
    return jnp.sum(out[:, 0, 0])


# speedup vs baseline: 2.1329x; 1.2112x over previous
import jax
import jax.numpy as jnp
import numpy as np
from jax.experimental import pallas as pl
from jax.experimental.pallas import tpu as pltpu

N_RBF = 16
N_HIDDEN = 32
CUTOFF = 5.0
N_ATOMS = 2048
BLOCK = 256
NB = N_ATOMS // BLOCK
NPAIR = NB * (NB + 1) // 2


def _rbf_consts():
    c = np.linspace(0.0, CUTOFF, N_RBF, dtype=np.float32).astype(np.float64)
    width32 = np.float32(np.float32(c[1]) - np.float32(c[0]))
    coeff = np.float64(np.float32(-0.5 / (width32 * width32)))
    w = (c[N_RBF - 1] - c[0]) / (N_RBF - 1)
    a = -2.0 * coeff * w
    bases = [c[0], c[8]]
    scales = np.array(
        [np.exp(coeff * (c[k] - bases[k // 8]) ** 2) for k in range(N_RBF)],
        dtype=np.float64,
    )
    uclamp = 87.0 / (7.0 * a)
    return (np.float32(coeff), np.float32(a), np.float32(uclamp),
            [np.float32(b) for b in bases], scales.astype(np.float32))


def _block_pair(t):
    bi = jnp.int32(0)
    for b in range(1, NB):
        start = b * NB - (b * (b - 1)) // 2
        bi = bi + (t >= start).astype(jnp.int32)
    row_start = bi * NB - (bi * (bi - 1)) // 2
    bj = t - row_start + bi
    return bi, bj


def _body(rowpos_ref, posT_ref, W1T_ref, b1_ref, W2T_ref, b2_ref,
          W3T_ref, b3_ref, out_ref, feat_ref):
    coeff, a, uclamp, bases, scales = _rbf_consts()
    t_id = pl.program_id(0)
    bi, bj = _block_pair(t_id)
    diag = bi == bj

    @pl.when(t_id == 0)
    def _init():
        feat_ref[...] = jnp.zeros((N_RBF, N_ATOMS), jnp.float32)

    cx = rowpos_ref[:, 0:1]
    cy = rowpos_ref[:, 1:2]
    cz = rowpos_ref[:, 2:3]
    rx = posT_ref[0:1, :]
    ry = posT_ref[1:2, :]
    rz = posT_ref[2:3, :]

    dx = cx - rx
    dy = cy - ry
    dz = cz - rz
    d2 = dx * dx + dy * dy + dz * dz
    dist = jnp.sqrt(d2)
    mask = (d2 > 0.0) & (dist < CUTOFF)

    rows = []
    cols = []
    for g in range(2):
        u = jnp.minimum(dist - bases[g], uclamp)
        s = jnp.where(mask, jnp.exp(coeff * (u * u)), 0.0)
        tt = jnp.exp(a * u)
        p = s
        for m in range(8):
            if m:
                p = p * tt
            sc = float(scales[8 * g + m])
            rows.append(jnp.sum(p, axis=1) * sc)
            cols.append(jnp.sum(p, axis=0) * sc)
    row_tile = jnp.stack(rows, axis=0)
    col_tile = jnp.stack(cols, axis=0)

    sl_i = pl.ds(bi * BLOCK, BLOCK)
    feat_ref[:, sl_i] = feat_ref[:, sl_i] + row_tile

    @pl.when(jnp.logical_not(diag))
    def _offdiag():
        sl_j = pl.ds(bj * BLOCK, BLOCK)
        feat_ref[:, sl_j] = feat_ref[:, sl_j] + col_tile

    @pl.when(t_id == NPAIR - 1)
    def _mlp():
        feat = feat_ref[...]
        h = jax.lax.dot_general(W1T_ref[...], feat, (((1,), (0,)), ((), ())),
                                preferred_element_type=jnp.float32)
        h = jax.nn.silu(h + b1_ref[...])
        h = jax.lax.dot_general(W2T_ref[...], h, (((1,), (0,)), ((), ())),
                                preferred_element_type=jnp.float32)
        h = jax.nn.silu(h + b2_ref[...])
        e = jax.lax.dot_general(W3T_ref[...], h, (((1,), (0,)), ((), ())),
                                preferred_element_type=jnp.float32)
        energy = jnp.sum(e + b3_ref[...])
        out_ref[...] = jnp.broadcast_to(energy, (1, 128))


def kernel(positions, W1, b1, W2, b2, W3, b3):
    f32 = jnp.float32
    positions = positions.astype(f32)
    rowpos = jnp.zeros((N_ATOMS, 128), f32).at[:, :3].set(positions)
    posT = jnp.zeros((8, N_ATOMS), f32).at[:3, :].set(positions.T)

    W1T = W1.T.astype(f32)
    W2T = W2.T.astype(f32)
    W3T = W3.T.astype(f32)
    b1c = b1.astype(f32)[:, None]
    b2c = b2.astype(f32)[:, None]
    b3c = b3.astype(f32)[:, None]

    out = pl.pallas_call(
        _body,
        grid=(NPAIR,),
        in_specs=[
            pl.BlockSpec((BLOCK, 128), lambda t: (_block_pair(t)[0], 0)),
            pl.BlockSpec((8, BLOCK), lambda t: (0, _block_pair(t)[1])),
            pl.BlockSpec(W1T.shape, lambda t: (0, 0)),
            pl.BlockSpec(b1c.shape, lambda t: (0, 0)),
            pl.BlockSpec(W2T.shape, lambda t: (0, 0)),
            pl.BlockSpec(b2c.shape, lambda t: (0, 0)),
            pl.BlockSpec(W3T.shape, lambda t: (0, 0)),
            pl.BlockSpec(b3c.shape, lambda t: (0, 0)),
        ],
        out_specs=pl.BlockSpec((1, 128), lambda t: (0, 0)),
        out_shape=jax.ShapeDtypeStruct((1, 128), f32),
        scratch_shapes=[pltpu.VMEM((N_RBF, N_ATOMS), f32)],
        compiler_params=pltpu.CompilerParams(
            dimension_semantics=("arbitrary",),
        ),
    )(rowpos, posT, W1T, b1c, W2T, b2c, W3T, b3c)
    return out[0, 0]
